# baseline (speedup 1.0000x reference)
import functools

import jax
import jax.numpy as jnp
from jax import lax
from jax.experimental import pallas as pl
from jax.experimental.pallas import tpu as pltpu
from jax.experimental.pallas import tpu_sc as plsc

VOCAB = 30522
HIDDEN = 768
MAXPOS = 512
B = 128
L = 512
EPS = 1e-12

NC, NS, LANES = 2, 16, 16
NW = NC * NS
TOK = B * L
TPW = TOK // NW
CH = 16
NCHUNK = TPW // CH
KV = HIDDEN // LANES
CPS = L // CH


def _allsum(v):
    for d in (8, 4, 2, 1):
        idx = lax.iota(jnp.int32, LANES) ^ d
        v = v + v.at[idx].get(mode="promise_in_bounds")
    return v


def _rsqrt_vec(v):
    i = plsc.bitcast(v, jnp.int32)
    y = plsc.bitcast(jnp.full((LANES,), 0x5F3759DF, dtype=jnp.int32)
                     - (i >> 1), jnp.float32)
    for _ in range(3):
        y = y * (1.5 - 0.5 * v * y * y)
    return y


def _body(ids_hbm, tids_hbm, word_hbm, pos_hbm, type_hbm, gamma_hbm, beta_hbm,
          out_hbm, idx_v, tidx_v, wbuf, pbuf, tbuf, gbuf, bbuf, sem_w, sem_t):
    wid = lax.axis_index("s") * NC + lax.axis_index("c")
    base0 = wid * TPW

    pltpu.sync_copy(gamma_hbm, gbuf)
    pltpu.sync_copy(beta_hbm, bbuf)

    def chunk(c, carry):
        base = base0 + c * CH
        posrow = (c % CPS) * CH
        pltpu.sync_copy(ids_hbm.at[pl.ds(base, CH)], idx_v)
        pltpu.sync_copy(tids_hbm.at[pl.ds(base, CH)], tidx_v)
        cp_w = pltpu.async_copy(word_hbm.at[idx_v], wbuf, sem_w)
        cp_t = pltpu.async_copy(type_hbm.at[tidx_v], tbuf, sem_t)
        pltpu.sync_copy(pos_hbm.at[pl.ds(posrow, CH)], pbuf)
        cp_w.wait()
        cp_t.wait()

        def token(i, carry2):
            acc = jnp.zeros((LANES,), jnp.float32)
            acc2 = jnp.zeros((LANES,), jnp.float32)
            for k in range(KV):
                sl = pl.ds(k * LANES, LANES)
                y = wbuf[i, sl] + pbuf[i, sl] + tbuf[i, sl]
                wbuf[i, sl] = y
                acc = acc + y
                acc2 = acc2 + y * y
            mvec = _allsum(acc) * (1.0 / HIDDEN)
            var = _allsum(acc2) * (1.0 / HIDDEN) - mvec * mvec
            inv = _rsqrt_vec(var + EPS)
            for k in range(KV):
                sl = pl.ds(k * LANES, LANES)
                o = (wbuf[i, sl] - mvec) * inv
                wbuf[i, sl] = o * gbuf[sl] + bbuf[sl]
            return carry2

        lax.fori_loop(0, CH, token, 0)
        pltpu.sync_copy(wbuf, out_hbm.at[pl.ds(base, CH)])
        return carry

    lax.fori_loop(0, NCHUNK, chunk, 0)


_emb_call = pl.kernel(
    _body,
    out_type=jax.ShapeDtypeStruct((TOK, HIDDEN), jnp.float32),
    mesh=plsc.VectorSubcoreMesh(core_axis_name="c", subcore_axis_name="s",
                                num_cores=NC, num_subcores=NS),
    scratch_types=[
        pltpu.VMEM((CH,), jnp.int32),
        pltpu.VMEM((CH,), jnp.int32),
        pltpu.VMEM((CH, HIDDEN), jnp.float32),
        pltpu.VMEM((CH, HIDDEN), jnp.float32),
        pltpu.VMEM((CH, HIDDEN), jnp.float32),
        pltpu.VMEM((HIDDEN,), jnp.float32),
        pltpu.VMEM((HIDDEN,), jnp.float32),
        pltpu.SemaphoreType.DMA,
        pltpu.SemaphoreType.DMA,
    ],
    compiler_params=pltpu.CompilerParams(needs_layout_passes=False),
)


def kernel(input_ids, token_type_ids, word_emb, pos_emb, type_emb,
           ln_gamma, ln_beta):
    ids = input_ids.reshape(-1).astype(jnp.int32)
    tids = token_type_ids.reshape(-1).astype(jnp.int32)
    out = _emb_call(ids, tids, word_emb, pos_emb, type_emb, ln_gamma, ln_beta)
    return out.reshape(B, L, HIDDEN)


# speedup vs baseline: 2.5520x; 2.5520x over previous
import jax
import jax.numpy as jnp
from jax import lax
from jax.experimental import pallas as pl
from jax.experimental.pallas import tpu as pltpu
from jax.experimental.pallas import tpu_sc as plsc

VOCAB = 30522
HIDDEN = 768
MAXPOS = 512
B = 128
L = 512
EPS = 1e-12

NC, NS, LANES = 2, 16, 16
NW = NC * NS
TOK = B * L
TPW = TOK // NW
CH = 16
NCHUNK = TPW // CH
KV = HIDDEN // LANES
CPS = L // CH
NPT = 2 * MAXPOS
PT_PER_TILE = NPT // NS
P_PER_TILE = MAXPOS // NS
PB = 8


def _allsum(v):
    for d in (8, 4, 2, 1):
        idx = lax.iota(jnp.int32, LANES) ^ d
        v = v + v.at[idx].get(mode="promise_in_bounds")
    return v


def _rsqrt_vec(v):
    i = plsc.bitcast(v, jnp.int32)
    y = plsc.bitcast(jnp.full((LANES,), 0x5F3759DF, dtype=jnp.int32)
                     - (i >> 1), jnp.float32)
    for _ in range(3):
        y = y * (1.5 - 0.5 * v * y * y)
    return y


def _body(ids_hbm, tids_hbm, word_hbm, pos_hbm, type_hbm, gb_hbm,
          out_hbm, pt_hbm,
          ids_v, tids_v, gb_v, ptidx_v, wbuf, ptbuf, pbld, obld, tbld,
          sem_g, sem_o, sem_b):
    cid = lax.axis_index("c")
    sid = lax.axis_index("s")
    wid = sid * NC + cid
    base0 = wid * TPW

    pltpu.sync_copy(type_hbm, tbld)
    p0 = sid * P_PER_TILE

    def build(j, carry):
        pj = p0 + j * PB
        pltpu.sync_copy(pos_hbm.at[pl.ds(pj, PB)], pbld)

        def brow(i, c2):
            for tt in range(2):
                for k in range(KV):
                    sl = pl.ds(k * LANES, LANES)
                    obld[2 * i + tt, sl] = pbld[i, sl] + tbld[tt, sl]
            return c2

        lax.fori_loop(0, PB, brow, 0)
        pltpu.async_copy(obld, pt_hbm.at[cid, pl.ds(2 * pj, 2 * PB)],
                         sem_b).wait()
        return carry

    lax.fori_loop(0, P_PER_TILE // PB, build, 0)
    plsc.subcore_barrier()

    pltpu.sync_copy(ids_hbm.at[pl.ds(base0, TPW)], ids_v)
    pltpu.sync_copy(tids_hbm.at[pl.ds(base0, TPW)], tids_v)
    pltpu.sync_copy(gb_hbm, gb_v)

    def chunk(c, carry):
        base = base0 + c * CH
        posrow = (c % CPS) * CH
        tid = tids_v[pl.ds(c * CH, CH)]
        ptidx_v[...] = 2 * posrow + 2 * lax.iota(jnp.int32, CH) + tid
        cp_w = pltpu.async_copy(word_hbm.at[ids_v.at[pl.ds(c * CH, CH)]],
                                wbuf, sem_g)
        cp_p = pltpu.async_copy(pt_hbm.at[cid].at[ptidx_v], ptbuf, sem_g)
        cp_w.wait()
        cp_p.wait()

        def token(i, carry2):
            acc = jnp.zeros((LANES,), jnp.float32)
            acc2 = jnp.zeros((LANES,), jnp.float32)
            ys = []
            for k in range(KV):
                sl = pl.ds(k * LANES, LANES)
                y = wbuf[i, sl] + ptbuf[i, sl]
                ys.append(y)
                acc = acc + y
                acc2 = acc2 + y * y
            mvec = _allsum(acc) * (1.0 / HIDDEN)
            var = _allsum(acc2) * (1.0 / HIDDEN) - mvec * mvec
            inv = _rsqrt_vec(var + EPS)
            for k in range(KV):
                sl = pl.ds(k * LANES, LANES)
                g = gb_v[pl.ds(k * LANES, LANES)]
                bta = gb_v[pl.ds((KV + k) * LANES, LANES)]
                wbuf[i, sl] = (ys[k] - mvec) * inv * g + bta
            return carry2

        lax.fori_loop(0, CH, token, 0)
        pltpu.async_copy(wbuf, out_hbm.at[pl.ds(base, CH)], sem_o).wait()
        return carry

    lax.fori_loop(0, NCHUNK, chunk, 0)


_emb_call = pl.kernel(
    _body,
    out_type=(
        jax.ShapeDtypeStruct((TOK, HIDDEN), jnp.float32),
        jax.ShapeDtypeStruct((NC, NPT, HIDDEN), jnp.float32),
    ),
    mesh=plsc.VectorSubcoreMesh(core_axis_name="c", subcore_axis_name="s",
                                num_cores=NC, num_subcores=NS),
    scratch_types=[
        pltpu.VMEM((TPW,), jnp.int32),
        pltpu.VMEM((TPW,), jnp.int32),
        pltpu.VMEM((2 * HIDDEN,), jnp.float32),
        pltpu.VMEM((CH,), jnp.int32),
        pltpu.VMEM((CH, HIDDEN), jnp.float32),
        pltpu.VMEM((CH, HIDDEN), jnp.float32),
        pltpu.VMEM((PB, HIDDEN), jnp.float32),
        pltpu.VMEM((2 * PB, HIDDEN), jnp.float32),
        pltpu.VMEM((2, HIDDEN), jnp.float32),
        pltpu.SemaphoreType.DMA,
        pltpu.SemaphoreType.DMA,
        pltpu.SemaphoreType.DMA,
    ],
    compiler_params=pltpu.CompilerParams(needs_layout_passes=False),
)


def kernel(input_ids, token_type_ids, word_emb, pos_emb, type_emb,
           ln_gamma, ln_beta):
    ids = input_ids.reshape(-1).astype(jnp.int32)
    tids = token_type_ids.reshape(-1).astype(jnp.int32)
    gb = jnp.concatenate([ln_gamma, ln_beta])
    out, _ = _emb_call(ids, tids, word_emb, pos_emb, type_emb, gb)
    return out.reshape(B, L, HIDDEN)


# speedup vs baseline: 6.6997x; 2.6253x over previous
import jax
import jax.numpy as jnp
from jax import lax
from jax.experimental import pallas as pl
from jax.experimental.pallas import tpu as pltpu
from jax.experimental.pallas import tpu_sc as plsc

VOCAB = 30522
HIDDEN = 768
MAXPOS = 512
B = 128
L = 512
EPS = 1e-12

NC, NS = 2, 16
NW = NC * NS
TOK = B * L
TPW = TOK // NW
CH = 64
NCHUNK = TPW // CH

LB = 256
NJ = L // LB


def _gather_body(ids_hbm, word_hbm, y_hbm,
                 ids_v, buf_a, buf_b, sem_a, sem_b, sem_oa, sem_ob):
    wid = lax.axis_index("s") * NC + lax.axis_index("c")
    base0 = wid * TPW
    pltpu.sync_copy(ids_hbm.at[pl.ds(base0, TPW)], ids_v)

    slots = ((buf_a, sem_a, sem_oa), (buf_b, sem_b, sem_ob))

    def issue(c, buf, sem_g):
        pltpu.async_copy(word_hbm.at[ids_v.at[pl.ds(c * CH, CH)]], buf, sem_g)

    for s in range(2):
        issue(s, slots[s][0], slots[s][1])

    def half(h, carry):
        for s in range(2):
            buf, sem_g, sem_o = slots[s]
            c = 2 * h + s
            pltpu.make_async_copy(word_hbm.at[pl.ds(0, CH)], buf,
                                  sem_g).wait()
            pltpu.async_copy(buf, y_hbm.at[pl.ds(base0 + c * CH, CH)], sem_o)
            pltpu.make_async_copy(buf, y_hbm.at[pl.ds(0, CH)], sem_o).wait()
            cn = jnp.minimum(c + 2, NCHUNK - 1)
            issue(cn, buf, sem_g)
        return carry

    lax.fori_loop(0, NCHUNK // 2, half, 0)
    for s in range(2):
        buf, sem_g, sem_o = slots[s]
        pltpu.make_async_copy(word_hbm.at[pl.ds(0, CH)], buf, sem_g).wait()


_gather_call = pl.kernel(
    _gather_body,
    out_type=jax.ShapeDtypeStruct((TOK, HIDDEN), jnp.float32),
    mesh=plsc.VectorSubcoreMesh(core_axis_name="c", subcore_axis_name="s",
                                num_cores=NC, num_subcores=NS),
    scratch_types=[
        pltpu.VMEM((TPW,), jnp.int32),
        pltpu.VMEM((CH, HIDDEN), jnp.float32),
        pltpu.VMEM((CH, HIDDEN), jnp.float32),
        pltpu.SemaphoreType.DMA,
        pltpu.SemaphoreType.DMA,
        pltpu.SemaphoreType.DMA,
        pltpu.SemaphoreType.DMA,
    ],
    compiler_params=pltpu.CompilerParams(needs_layout_passes=False),
)


def _ln_body(y_ref, tt_ref, pos_ref, type_ref, gamma_ref, beta_ref, out_ref):
    j = pl.program_id(0)
    x = y_ref[0] + pos_ref[...]
    ttf = tt_ref[0, 0, pl.ds(j * LB, LB)]
    t0 = type_ref[0, :]
    dt = type_ref[1, :] - t0
    x = x + t0[None, :] + ttf[:, None] * dt[None, :]
    mean = jnp.mean(x, axis=-1, keepdims=True)
    var = jnp.mean(jnp.square(x - mean), axis=-1, keepdims=True)
    x = (x - mean) * lax.rsqrt(var + EPS)
    out_ref[0] = x * gamma_ref[...] + beta_ref[...]


_ln_call = pl.pallas_call(
    _ln_body,
    grid=(NJ, B),
    in_specs=[
        pl.BlockSpec((1, LB, HIDDEN), lambda j, b: (b, j, 0)),
        pl.BlockSpec((1, 1, L), lambda j, b: (b, 0, 0)),
        pl.BlockSpec((LB, HIDDEN), lambda j, b: (j, 0)),
        pl.BlockSpec((2, HIDDEN), lambda j, b: (0, 0)),
        pl.BlockSpec((HIDDEN,), lambda j, b: (0,)),
        pl.BlockSpec((HIDDEN,), lambda j, b: (0,)),
    ],
    out_specs=pl.BlockSpec((1, LB, HIDDEN), lambda j, b: (b, j, 0)),
    out_shape=jax.ShapeDtypeStruct((B, L, HIDDEN), jnp.float32),
    compiler_params=pltpu.CompilerParams(
        dimension_semantics=("arbitrary", "arbitrary")),
)


def kernel(input_ids, token_type_ids, word_emb, pos_emb, type_emb,
           ln_gamma, ln_beta):
    ids = input_ids.reshape(-1).astype(jnp.int32)
    ttf = token_type_ids.astype(jnp.float32).reshape(B, 1, L)
    y = _gather_call(ids, word_emb)
    y = y.reshape(B, L, HIDDEN)
    return _ln_call(y, ttf, pos_emb, type_emb, ln_gamma, ln_beta)


# speedup vs baseline: 8.0617x; 1.2033x over previous
import jax
import jax.numpy as jnp
from jax import lax
from jax.experimental import pallas as pl
from jax.experimental.pallas import tpu as pltpu
from jax.experimental.pallas import tpu_sc as plsc

VOCAB = 30522
HIDDEN = 768
MAXPOS = 512
B = 128
L = 512
EPS = 1e-12

NC, NS = 2, 16
NW = NC * NS
TOK = B * L
TPW = TOK // NW
CH = 64
NCHUNK = TPW // CH

LB = 512
NJ = L // LB


def _gather_body(ids_hbm, word_hbm, y_hbm,
                 ids_v, buf_a, buf_b, sem_a, sem_b, sem_oa, sem_ob):
    wid = lax.axis_index("s") * NC + lax.axis_index("c")
    base0 = wid * TPW
    pltpu.sync_copy(ids_hbm.at[pl.ds(base0, TPW)], ids_v)

    slots = ((buf_a, sem_a, sem_oa), (buf_b, sem_b, sem_ob))

    def issue(c, buf, sem_g):
        pltpu.async_copy(word_hbm.at[ids_v.at[pl.ds(c * CH, CH)]], buf, sem_g)

    for s in range(2):
        issue(s, slots[s][0], slots[s][1])

    def half(h, carry):
        for s in range(2):
            buf, sem_g, sem_o = slots[s]
            c = 2 * h + s
            pltpu.make_async_copy(word_hbm.at[pl.ds(0, CH)], buf,
                                  sem_g).wait()
            pltpu.async_copy(buf, y_hbm.at[pl.ds(base0 + c * CH, CH)], sem_o)
            pltpu.make_async_copy(buf, y_hbm.at[pl.ds(0, CH)], sem_o).wait()
            cn = jnp.minimum(c + 2, NCHUNK - 1)
            issue(cn, buf, sem_g)
        return carry

    lax.fori_loop(0, NCHUNK // 2, half, 0)
    for s in range(2):
        buf, sem_g, sem_o = slots[s]
        pltpu.make_async_copy(word_hbm.at[pl.ds(0, CH)], buf, sem_g).wait()


_gather_call = pl.kernel(
    _gather_body,
    out_type=jax.ShapeDtypeStruct((TOK, HIDDEN), jnp.float32),
    mesh=plsc.VectorSubcoreMesh(core_axis_name="c", subcore_axis_name="s",
                                num_cores=NC, num_subcores=NS),
    scratch_types=[
        pltpu.VMEM((TPW,), jnp.int32),
        pltpu.VMEM((CH, HIDDEN), jnp.float32),
        pltpu.VMEM((CH, HIDDEN), jnp.float32),
        pltpu.SemaphoreType.DMA,
        pltpu.SemaphoreType.DMA,
        pltpu.SemaphoreType.DMA,
        pltpu.SemaphoreType.DMA,
    ],
    compiler_params=pltpu.CompilerParams(needs_layout_passes=False),
)


def _ln_body(y_ref, tt_ref, pos_ref, type_ref, gamma_ref, beta_ref, out_ref):
    x = y_ref[0] + pos_ref[...]
    ttf = tt_ref[0, 0, :]
    t0 = type_ref[0, :]
    dt = type_ref[1, :] - t0
    x = x + t0[None, :] + ttf[:, None] * dt[None, :]
    mean = jnp.mean(x, axis=-1, keepdims=True)
    var = jnp.mean(jnp.square(x - mean), axis=-1, keepdims=True)
    x = (x - mean) * lax.rsqrt(var + EPS)
    out_ref[0] = x * gamma_ref[...] + beta_ref[...]


_ln_call = pl.pallas_call(
    _ln_body,
    grid=(B,),
    in_specs=[
        pl.BlockSpec((1, LB, HIDDEN), lambda b: (b, 0, 0)),
        pl.BlockSpec((1, 1, L), lambda b: (b, 0, 0)),
        pl.BlockSpec((LB, HIDDEN), lambda b: (0, 0)),
        pl.BlockSpec((2, HIDDEN), lambda b: (0, 0)),
        pl.BlockSpec((HIDDEN,), lambda b: (0,)),
        pl.BlockSpec((HIDDEN,), lambda b: (0,)),
    ],
    out_specs=pl.BlockSpec((1, LB, HIDDEN), lambda b: (b, 0, 0)),
    out_shape=jax.ShapeDtypeStruct((B, L, HIDDEN), jnp.float32),
    compiler_params=pltpu.CompilerParams(
        dimension_semantics=("arbitrary",)),
)


def kernel(input_ids, token_type_ids, word_emb, pos_emb, type_emb,
           ln_gamma, ln_beta):
    ids = input_ids.reshape(-1).astype(jnp.int32)
    ttf = token_type_ids.astype(jnp.float32).reshape(B, 1, L)
    y = _gather_call(ids, word_emb)
    y = y.reshape(B, L, HIDDEN)
    return _ln_call(y, ttf, pos_emb, type_emb, ln_gamma, ln_beta)


# speedup vs baseline: 8.9775x; 1.1136x over previous
import jax
import jax.numpy as jnp
from jax import lax
from jax.experimental import pallas as pl
from jax.experimental.pallas import tpu as pltpu
from jax.experimental.pallas import tpu_sc as plsc

VOCAB = 30522
HIDDEN = 768
MAXPOS = 512
B = 128
L = 512
EPS = 1e-12

NC, NS = 2, 16
NW = NC * NS
TOK = B * L
TPW = TOK // NW
CH = 64
NCHUNK = TPW // CH

LB = 512
NJ = L // LB
BB = 2


def _gather_body(ids_hbm, word_hbm, y_hbm,
                 ids_v, buf_a, buf_b, sem_a, sem_b, sem_oa, sem_ob):
    wid = lax.axis_index("s") * NC + lax.axis_index("c")
    base0 = wid * TPW
    pltpu.sync_copy(ids_hbm.at[pl.ds(base0, TPW)], ids_v)

    slots = ((buf_a, sem_a, sem_oa), (buf_b, sem_b, sem_ob))

    def issue(c, buf, sem_g):
        pltpu.async_copy(word_hbm.at[ids_v.at[pl.ds(c * CH, CH)]], buf, sem_g)

    for s in range(2):
        issue(s, slots[s][0], slots[s][1])

    def half(h, carry):
        for s in range(2):
            buf, sem_g, sem_o = slots[s]
            c = 2 * h + s
            pltpu.make_async_copy(word_hbm.at[pl.ds(0, CH)], buf,
                                  sem_g).wait()
            pltpu.async_copy(buf, y_hbm.at[pl.ds(base0 + c * CH, CH)], sem_o)
            pltpu.make_async_copy(buf, y_hbm.at[pl.ds(0, CH)], sem_o).wait()
            cn = jnp.minimum(c + 2, NCHUNK - 1)
            issue(cn, buf, sem_g)
        return carry

    lax.fori_loop(0, NCHUNK // 2, half, 0)
    for s in range(2):
        buf, sem_g, sem_o = slots[s]
        pltpu.make_async_copy(word_hbm.at[pl.ds(0, CH)], buf, sem_g).wait()


_gather_call = pl.kernel(
    _gather_body,
    out_type=jax.ShapeDtypeStruct((TOK, HIDDEN), jnp.float32),
    mesh=plsc.VectorSubcoreMesh(core_axis_name="c", subcore_axis_name="s",
                                num_cores=NC, num_subcores=NS),
    scratch_types=[
        pltpu.VMEM((TPW,), jnp.int32),
        pltpu.VMEM((CH, HIDDEN), jnp.float32),
        pltpu.VMEM((CH, HIDDEN), jnp.float32),
        pltpu.SemaphoreType.DMA,
        pltpu.SemaphoreType.DMA,
        pltpu.SemaphoreType.DMA,
        pltpu.SemaphoreType.DMA,
    ],
    compiler_params=pltpu.CompilerParams(needs_layout_passes=False),
)


def _ln_body(y_ref, tt_ref, pos_ref, type_ref, gamma_ref, beta_ref, out_ref):
    for bb in range(BB):
        x = y_ref[bb] + pos_ref[...]
        ttf = tt_ref[bb, 0, :]
        t0 = type_ref[0, :]
        dt = type_ref[1, :] - t0
        x = x + t0[None, :] + ttf[:, None] * dt[None, :]
        mean = jnp.mean(x, axis=-1, keepdims=True)
        var = jnp.mean(jnp.square(x - mean), axis=-1, keepdims=True)
        x = (x - mean) * lax.rsqrt(var + EPS)
        out_ref[bb] = x * gamma_ref[...] + beta_ref[...]


_ln_call = pl.pallas_call(
    _ln_body,
    grid=(B // BB,),
    in_specs=[
        pl.BlockSpec((BB, LB, HIDDEN), lambda b: (b, 0, 0)),
        pl.BlockSpec((BB, 1, L), lambda b: (b, 0, 0)),
        pl.BlockSpec((LB, HIDDEN), lambda b: (0, 0)),
        pl.BlockSpec((2, HIDDEN), lambda b: (0, 0)),
        pl.BlockSpec((HIDDEN,), lambda b: (0,)),
        pl.BlockSpec((HIDDEN,), lambda b: (0,)),
    ],
    out_specs=pl.BlockSpec((BB, LB, HIDDEN), lambda b: (b, 0, 0)),
    out_shape=jax.ShapeDtypeStruct((B, L, HIDDEN), jnp.float32),
    compiler_params=pltpu.CompilerParams(
        dimension_semantics=("arbitrary",)),
)


def kernel(input_ids, token_type_ids, word_emb, pos_emb, type_emb,
           ln_gamma, ln_beta):
    ids = input_ids.reshape(-1).astype(jnp.int32)
    ttf = token_type_ids.astype(jnp.float32).reshape(B, 1, L)
    y = _gather_call(ids, word_emb)
    y = y.reshape(B, L, HIDDEN)
    return _ln_call(y, ttf, pos_emb, type_emb, ln_gamma, ln_beta)


# speedup vs baseline: 9.3726x; 1.0440x over previous
import jax
import jax.numpy as jnp
from jax import lax
from jax.experimental import pallas as pl
from jax.experimental.pallas import tpu as pltpu
from jax.experimental.pallas import tpu_sc as plsc

VOCAB = 30522
HIDDEN = 768
MAXPOS = 512
B = 128
L = 512
EPS = 1e-12

NC, NS = 2, 16
NW = NC * NS
TOK = B * L
TPW = TOK // NW
CH = 64
NCHUNK = TPW // CH

LB = 512
NJ = L // LB
BB = 4


def _gather_body(ids_hbm, word_hbm, y_hbm,
                 ids_v, buf_a, buf_b, sem_a, sem_b, sem_oa, sem_ob):
    wid = lax.axis_index("s") * NC + lax.axis_index("c")
    base0 = wid * TPW
    pltpu.sync_copy(ids_hbm.at[pl.ds(base0, TPW)], ids_v)

    slots = ((buf_a, sem_a, sem_oa), (buf_b, sem_b, sem_ob))

    def issue(c, buf, sem_g):
        pltpu.async_copy(word_hbm.at[ids_v.at[pl.ds(c * CH, CH)]], buf, sem_g)

    for s in range(2):
        issue(s, slots[s][0], slots[s][1])

    def half(h, carry):
        for s in range(2):
            buf, sem_g, sem_o = slots[s]
            c = 2 * h + s
            pltpu.make_async_copy(word_hbm.at[pl.ds(0, CH)], buf,
                                  sem_g).wait()
            pltpu.async_copy(buf, y_hbm.at[pl.ds(base0 + c * CH, CH)], sem_o)
            pltpu.make_async_copy(buf, y_hbm.at[pl.ds(0, CH)], sem_o).wait()
            cn = jnp.minimum(c + 2, NCHUNK - 1)
            issue(cn, buf, sem_g)
        return carry

    lax.fori_loop(0, NCHUNK // 2, half, 0)
    for s in range(2):
        buf, sem_g, sem_o = slots[s]
        pltpu.make_async_copy(word_hbm.at[pl.ds(0, CH)], buf, sem_g).wait()


_gather_call = pl.kernel(
    _gather_body,
    out_type=jax.ShapeDtypeStruct((TOK, HIDDEN), jnp.float32),
    mesh=plsc.VectorSubcoreMesh(core_axis_name="c", subcore_axis_name="s",
                                num_cores=NC, num_subcores=NS),
    scratch_types=[
        pltpu.VMEM((TPW,), jnp.int32),
        pltpu.VMEM((CH, HIDDEN), jnp.float32),
        pltpu.VMEM((CH, HIDDEN), jnp.float32),
        pltpu.SemaphoreType.DMA,
        pltpu.SemaphoreType.DMA,
        pltpu.SemaphoreType.DMA,
        pltpu.SemaphoreType.DMA,
    ],
    compiler_params=pltpu.CompilerParams(needs_layout_passes=False),
)


def _ln_body(y_ref, tt_ref, pos_ref, type_ref, gamma_ref, beta_ref, out_ref):
    for bb in range(BB):
        x = y_ref[bb] + pos_ref[...]
        ttf = tt_ref[bb, 0, :]
        t0 = type_ref[0, :]
        dt = type_ref[1, :] - t0
        x = x + t0[None, :] + ttf[:, None] * dt[None, :]
        mean = jnp.mean(x, axis=-1, keepdims=True)
        var = jnp.mean(jnp.square(x - mean), axis=-1, keepdims=True)
        x = (x - mean) * lax.rsqrt(var + EPS)
        out_ref[bb] = x * gamma_ref[...] + beta_ref[...]


_ln_call = pl.pallas_call(
    _ln_body,
    grid=(B // BB,),
    in_specs=[
        pl.BlockSpec((BB, LB, HIDDEN), lambda b: (b, 0, 0)),
        pl.BlockSpec((BB, 1, L), lambda b: (b, 0, 0)),
        pl.BlockSpec((LB, HIDDEN), lambda b: (0, 0)),
        pl.BlockSpec((2, HIDDEN), lambda b: (0, 0)),
        pl.BlockSpec((HIDDEN,), lambda b: (0,)),
        pl.BlockSpec((HIDDEN,), lambda b: (0,)),
    ],
    out_specs=pl.BlockSpec((BB, LB, HIDDEN), lambda b: (b, 0, 0)),
    out_shape=jax.ShapeDtypeStruct((B, L, HIDDEN), jnp.float32),
    compiler_params=pltpu.CompilerParams(
        dimension_semantics=("arbitrary",)),
)


def kernel(input_ids, token_type_ids, word_emb, pos_emb, type_emb,
           ln_gamma, ln_beta):
    ids = input_ids.reshape(-1).astype(jnp.int32)
    ttf = token_type_ids.astype(jnp.float32).reshape(B, 1, L)
    y = _gather_call(ids, word_emb)
    y = y.reshape(B, L, HIDDEN)
    return _ln_call(y, ttf, pos_emb, type_emb, ln_gamma, ln_beta)


# speedup vs baseline: 9.5065x; 1.0143x over previous
import jax
import jax.numpy as jnp
from jax import lax
from jax.experimental import pallas as pl
from jax.experimental.pallas import tpu as pltpu
from jax.experimental.pallas import tpu_sc as plsc

VOCAB = 30522
HIDDEN = 768
MAXPOS = 512
B = 128
L = 512
EPS = 1e-12

NC, NS = 2, 16
NW = NC * NS
TOK = B * L
TPW = TOK // NW
CH = 64
NCHUNK = TPW // CH

LB = 512
NJ = L // LB
BB = 8


def _gather_body(ids_hbm, word_hbm, y_hbm,
                 ids_v, buf_a, buf_b, sem_a, sem_b, sem_oa, sem_ob):
    wid = lax.axis_index("s") * NC + lax.axis_index("c")
    base0 = wid * TPW
    pltpu.sync_copy(ids_hbm.at[pl.ds(base0, TPW)], ids_v)

    slots = ((buf_a, sem_a, sem_oa), (buf_b, sem_b, sem_ob))

    def issue(c, buf, sem_g):
        pltpu.async_copy(word_hbm.at[ids_v.at[pl.ds(c * CH, CH)]], buf, sem_g)

    for s in range(2):
        issue(s, slots[s][0], slots[s][1])

    def half(h, carry):
        for s in range(2):
            buf, sem_g, sem_o = slots[s]
            c = 2 * h + s
            pltpu.make_async_copy(word_hbm.at[pl.ds(0, CH)], buf,
                                  sem_g).wait()
            pltpu.async_copy(buf, y_hbm.at[pl.ds(base0 + c * CH, CH)], sem_o)
            pltpu.make_async_copy(buf, y_hbm.at[pl.ds(0, CH)], sem_o).wait()
            cn = jnp.minimum(c + 2, NCHUNK - 1)
            issue(cn, buf, sem_g)
        return carry

    lax.fori_loop(0, NCHUNK // 2, half, 0)
    for s in range(2):
        buf, sem_g, sem_o = slots[s]
        pltpu.make_async_copy(word_hbm.at[pl.ds(0, CH)], buf, sem_g).wait()


_gather_call = pl.kernel(
    _gather_body,
    out_type=jax.ShapeDtypeStruct((TOK, HIDDEN), jnp.float32),
    mesh=plsc.VectorSubcoreMesh(core_axis_name="c", subcore_axis_name="s",
                                num_cores=NC, num_subcores=NS),
    scratch_types=[
        pltpu.VMEM((TPW,), jnp.int32),
        pltpu.VMEM((CH, HIDDEN), jnp.float32),
        pltpu.VMEM((CH, HIDDEN), jnp.float32),
        pltpu.SemaphoreType.DMA,
        pltpu.SemaphoreType.DMA,
        pltpu.SemaphoreType.DMA,
        pltpu.SemaphoreType.DMA,
    ],
    compiler_params=pltpu.CompilerParams(needs_layout_passes=False),
)


def _ln_body(y_ref, tt_ref, pos_ref, type_ref, gamma_ref, beta_ref, out_ref):
    for bb in range(BB):
        x = y_ref[bb] + pos_ref[...]
        ttf = tt_ref[bb, 0, :]
        t0 = type_ref[0, :]
        dt = type_ref[1, :] - t0
        x = x + t0[None, :] + ttf[:, None] * dt[None, :]
        mean = jnp.mean(x, axis=-1, keepdims=True)
        var = jnp.mean(jnp.square(x - mean), axis=-1, keepdims=True)
        x = (x - mean) * lax.rsqrt(var + EPS)
        out_ref[bb] = x * gamma_ref[...] + beta_ref[...]


_ln_call = pl.pallas_call(
    _ln_body,
    grid=(B // BB,),
    in_specs=[
        pl.BlockSpec((BB, LB, HIDDEN), lambda b: (b, 0, 0)),
        pl.BlockSpec((BB, 1, L), lambda b: (b, 0, 0)),
        pl.BlockSpec((LB, HIDDEN), lambda b: (0, 0)),
        pl.BlockSpec((2, HIDDEN), lambda b: (0, 0)),
        pl.BlockSpec((HIDDEN,), lambda b: (0,)),
        pl.BlockSpec((HIDDEN,), lambda b: (0,)),
    ],
    out_specs=pl.BlockSpec((BB, LB, HIDDEN), lambda b: (b, 0, 0)),
    out_shape=jax.ShapeDtypeStruct((B, L, HIDDEN), jnp.float32),
    compiler_params=pltpu.CompilerParams(
        dimension_semantics=("arbitrary",)),
)


def kernel(input_ids, token_type_ids, word_emb, pos_emb, type_emb,
           ln_gamma, ln_beta):
    ids = input_ids.reshape(-1).astype(jnp.int32)
    ttf = token_type_ids.astype(jnp.float32).reshape(B, 1, L)
    y = _gather_call(ids, word_emb)
    y = y.reshape(B, L, HIDDEN)
    return _ln_call(y, ttf, pos_emb, type_emb, ln_gamma, ln_beta)
